# baseline (speedup 1.0000x reference)
import functools

import jax
import jax.numpy as jnp
from jax import lax
from jax.experimental import pallas as pl
from jax.experimental.pallas import tpu as pltpu
from jax.experimental.pallas import tpu_sc as plsc

B = 16384
L = 50
D = 64
RING = 4


def _make_kernel(n_workers):
    bags_per_w = B // n_workers
    mesh = plsc.VectorSubcoreMesh(core_axis_name="c", subcore_axis_name="s")

    @functools.partial(
        pl.kernel,
        mesh=mesh,
        out_type=jax.ShapeDtypeStruct((B, D), jnp.bfloat16),
        compiler_params=pltpu.CompilerParams(
            use_tc_tiling_on_sc=False, needs_layout_passes=False
        ),
        scratch_types=[
            pltpu.VMEM((bags_per_w * 128,), jnp.int32),
            pltpu.VMEM((RING, L, D), jnp.bfloat16),
            pltpu.VMEM((bags_per_w, D), jnp.bfloat16),
        ]
        + [pltpu.SemaphoreType.DMA] * RING,
    )
    def embag(idx_hbm, w_hbm, out_hbm, idx_v, rows_v, out_v, *sems):
        n_cores = lax.axis_size("c")
        wid = lax.axis_index("s") * n_cores + lax.axis_index("c")

        pltpu.sync_copy(
            idx_hbm.at[pl.ds(wid * bags_per_w * 128, bags_per_w * 128)], idx_v
        )

        for b in range(RING):
            pltpu.async_copy(
                w_hbm.at[idx_v.at[pl.ds(b * 128, L)]], rows_v.at[b], sems[b]
            )

        def group_body(p, _):
            for b in range(RING):
                c = p * RING + b
                pltpu.make_async_copy(
                    w_hbm.at[idx_v.at[pl.ds(c * 128, L)]],
                    rows_v.at[b], sems[b],
                ).wait()
                acc = []
                for h in range(D // 32):
                    x = rows_v[b, 0, pl.ds(h * 32, 32)]
                    acc.extend(plsc.unpack(x, format=plsc.PackFormat.INTERLEAVED))
                for r in range(1, L):
                    for h in range(D // 32):
                        x = rows_v[b, r, pl.ds(h * 32, 32)]
                        ev, od = plsc.unpack(x, format=plsc.PackFormat.INTERLEAVED)
                        acc[2 * h] = acc[2 * h] + ev
                        acc[2 * h + 1] = acc[2 * h + 1] + od
                for h in range(D // 32):
                    out_v[c, pl.ds(h * 32, 32)] = plsc.pack(
                        acc[2 * h], acc[2 * h + 1],
                        format=plsc.PackFormat.INTERLEAVED,
                    )

                @pl.when(c + RING < bags_per_w)
                def _():
                    pltpu.async_copy(
                        w_hbm.at[idx_v.at[pl.ds((c + RING) * 128, L)]],
                        rows_v.at[b], sems[b],
                    )

            return ()

        lax.fori_loop(0, bags_per_w // RING, group_body, ())

        pltpu.sync_copy(
            out_v, out_hbm.at[pl.ds(wid * bags_per_w, bags_per_w), :]
        )

    return embag


@jax.jit
def kernel(indices, W):
    info = plsc.get_sparse_core_info()
    n_workers = info.num_cores * info.num_subcores
    idxp = jnp.pad(indices.astype(jnp.int32), ((0, 0), (0, 128 - L)))
    w16 = W.astype(jnp.bfloat16)
    out16 = _make_kernel(n_workers)(jnp.reshape(idxp, (-1,)), w16)
    return out16.astype(jnp.float32)


# speedup vs baseline: 1.1635x; 1.1635x over previous
import functools

import jax
import jax.numpy as jnp
from jax import lax
from jax.experimental import pallas as pl
from jax.experimental.pallas import tpu as pltpu
from jax.experimental.pallas import tpu_sc as plsc

B = 16384
L = 50
D = 64
RING = 8


def _make_kernel(n_workers):
    bags_per_w = B // n_workers
    mesh = plsc.VectorSubcoreMesh(core_axis_name="c", subcore_axis_name="s")

    @functools.partial(
        pl.kernel,
        mesh=mesh,
        out_type=jax.ShapeDtypeStruct((B, D), jnp.float32),
        compiler_params=pltpu.CompilerParams(use_tc_tiling_on_sc=False),
        scratch_types=[
            pltpu.VMEM((bags_per_w * 128,), jnp.int32),
            pltpu.VMEM((RING, L, D), jnp.float32),
            pltpu.VMEM((bags_per_w, D), jnp.float32),
        ]
        + [pltpu.SemaphoreType.DMA] * RING,
    )
    def embag(idx_hbm, w_hbm, out_hbm, idx_v, rows_v, out_v, *sems):
        n_cores = lax.axis_size("c")
        wid = lax.axis_index("s") * n_cores + lax.axis_index("c")

        pltpu.sync_copy(
            idx_hbm.at[pl.ds(wid * bags_per_w * 128, bags_per_w * 128)], idx_v
        )

        for b in range(RING):
            pltpu.async_copy(
                w_hbm.at[idx_v.at[pl.ds(b * 128, L)]], rows_v.at[b], sems[b]
            )

        def group_body(p, _):
            for b in range(RING):
                c = p * RING + b
                pltpu.make_async_copy(
                    w_hbm.at[idx_v.at[pl.ds(c * 128, L)]],
                    rows_v.at[b], sems[b],
                ).wait()
                acc = [rows_v[b, 0, pl.ds(d * 16, 16)] for d in range(D // 16)]
                for r in range(1, L):
                    for d in range(D // 16):
                        acc[d] = acc[d] + rows_v[b, r, pl.ds(d * 16, 16)]
                for d in range(D // 16):
                    out_v[c, pl.ds(d * 16, 16)] = acc[d]

                @pl.when(c + RING < bags_per_w)
                def _():
                    pltpu.async_copy(
                        w_hbm.at[idx_v.at[pl.ds((c + RING) * 128, L)]],
                        rows_v.at[b], sems[b],
                    )

            return ()

        lax.fori_loop(0, bags_per_w // RING, group_body, ())

        pltpu.sync_copy(
            out_v, out_hbm.at[pl.ds(wid * bags_per_w, bags_per_w), :]
        )

    return embag


@jax.jit
def kernel(indices, W):
    info = plsc.get_sparse_core_info()
    n_workers = info.num_cores * info.num_subcores
    idxp = jnp.pad(indices.astype(jnp.int32), ((0, 0), (0, 128 - L)))
    wt = lax.optimization_barrier(jnp.transpose(W))
    w_lin = jnp.transpose(wt)
    return _make_kernel(n_workers)(jnp.reshape(idxp, (-1,)), w_lin)


# speedup vs baseline: 1.2535x; 1.0774x over previous
import functools

import jax
import jax.numpy as jnp
from jax import lax
from jax.experimental import pallas as pl
from jax.experimental.pallas import tpu as pltpu
from jax.experimental.pallas import tpu_sc as plsc

B = 16384
L = 50
D = 64
RING = 4


def _make_kernel(n_workers):
    bags_per_w = B // n_workers
    mesh = plsc.VectorSubcoreMesh(core_axis_name="c", subcore_axis_name="s")

    @functools.partial(
        pl.kernel,
        mesh=mesh,
        out_type=jax.ShapeDtypeStruct((B, D), jnp.float32),
        compiler_params=pltpu.CompilerParams(use_tc_tiling_on_sc=False),
        scratch_types=[
            pltpu.VMEM((bags_per_w * 128,), jnp.int32),
            pltpu.VMEM((RING, L, D), jnp.float32),
            pltpu.VMEM((bags_per_w, D), jnp.float32),
        ]
        + [pltpu.SemaphoreType.DMA] * RING,
    )
    def embag(idx_hbm, w_hbm, out_hbm, idx_v, rows_v, out_v, *sems):
        n_cores = lax.axis_size("c")
        wid = lax.axis_index("s") * n_cores + lax.axis_index("c")

        pltpu.sync_copy(
            idx_hbm.at[pl.ds(wid * bags_per_w * 128, bags_per_w * 128)], idx_v
        )

        for b in range(RING):
            pltpu.async_copy(
                w_hbm.at[idx_v.at[pl.ds(b * 128, L)]], rows_v.at[b], sems[b]
            )

        def group_body(p, _):
            for b in range(RING):
                c = p * RING + b
                pltpu.make_async_copy(
                    w_hbm.at[idx_v.at[pl.ds(c * 128, L)]],
                    rows_v.at[b], sems[b],
                ).wait()
                acc = [rows_v[b, 0, pl.ds(d * 16, 16)] for d in range(D // 16)]
                for r in range(1, L):
                    for d in range(D // 16):
                        acc[d] = acc[d] + rows_v[b, r, pl.ds(d * 16, 16)]
                for d in range(D // 16):
                    out_v[c, pl.ds(d * 16, 16)] = acc[d]

                @pl.when(c + RING < bags_per_w)
                def _():
                    pltpu.async_copy(
                        w_hbm.at[idx_v.at[pl.ds((c + RING) * 128, L)]],
                        rows_v.at[b], sems[b],
                    )

            return ()

        lax.fori_loop(0, bags_per_w // RING, group_body, ())

        pltpu.sync_copy(
            out_v, out_hbm.at[pl.ds(wid * bags_per_w, bags_per_w), :]
        )

    return embag


@jax.jit
def kernel(indices, W):
    info = plsc.get_sparse_core_info()
    n_workers = info.num_cores * info.num_subcores
    idxp = jnp.pad(indices.astype(jnp.int32), ((0, 0), (0, 128 - L)))
    return _make_kernel(n_workers)(jnp.reshape(idxp, (-1,)), W)
